# Initial kernel scaffold; baseline (speedup 1.0000x reference)
import jax
import jax.numpy as jnp
from jax.experimental import pallas as pl


def kernel(feature, text_embedding, feature_mu, text_mu, combined_mu, temperature):
    raise NotImplementedError("write your pallas kernel here")


# speedup vs baseline: 5.4060x; 5.4060x over previous
import jax
import jax.numpy as jnp
from jax.experimental import pallas as pl
from jax.experimental.pallas import tpu as pltpu

_TILE = 1024


def _router_body(temp_ref, feat_ref, text_ref, fmu_ref, tmu_ref, cmu_ref,
                 onehot_ref, logits_ref):
    p1 = jnp.dot(feat_ref[...], fmu_ref[...], preferred_element_type=jnp.float32)
    p2 = jnp.dot(text_ref[...], tmu_ref[...], preferred_element_type=jnp.float32)
    combined = jnp.concatenate([p1, p2], axis=1)
    logits = jnp.dot(combined, cmu_ref[...], preferred_element_type=jnp.float32)

    eff_temp = jnp.clip(temp_ref[0], 0.5, 5.0)
    logits = jnp.clip(logits / eff_temp, -20.0, 20.0)
    logits_ref[...] = logits

    n = logits.shape[1]
    col = jax.lax.broadcasted_iota(jnp.int32, logits.shape, 1)
    row_max = jnp.max(logits, axis=1, keepdims=True)
    first_arg = jnp.min(jnp.where(logits == row_max, col, n), axis=1,
                        keepdims=True)
    onehot_ref[...] = (col == first_arg).astype(jnp.float32)


def kernel(feature, text_embedding, feature_mu, text_mu, combined_mu,
           temperature):
    tokens, dmodel = feature.shape
    nproj = feature_mu.shape[1]
    nexp = combined_mu.shape[1]
    grid = (tokens // _TILE,)

    onehot, logits = pl.pallas_call(
        _router_body,
        grid=grid,
        in_specs=[
            pl.BlockSpec(memory_space=pltpu.SMEM),
            pl.BlockSpec((_TILE, dmodel), lambda i: (i, 0)),
            pl.BlockSpec((_TILE, dmodel), lambda i: (i, 0)),
            pl.BlockSpec((dmodel, nproj), lambda i: (0, 0)),
            pl.BlockSpec((dmodel, nproj), lambda i: (0, 0)),
            pl.BlockSpec((2 * nproj, nexp), lambda i: (0, 0)),
        ],
        out_specs=[
            pl.BlockSpec((_TILE, nexp), lambda i: (i, 0)),
            pl.BlockSpec((_TILE, nexp), lambda i: (i, 0)),
        ],
        out_shape=[
            jax.ShapeDtypeStruct((tokens, nexp), jnp.float32),
            jax.ShapeDtypeStruct((tokens, nexp), jnp.float32),
        ],
        compiler_params=pltpu.CompilerParams(
            dimension_semantics=("arbitrary",),
        ),
    )(temperature, feature, text_embedding, feature_mu, text_mu, combined_mu)
    return (onehot, logits)


# speedup vs baseline: 5.8775x; 1.0872x over previous
import jax
import jax.numpy as jnp
from jax.experimental import pallas as pl
from jax.experimental.pallas import tpu as pltpu

_TILE = 2048


def _router_body(temp_ref, feat_ref, text_ref, fmu_ref, tmu_ref, cmu_ref,
                 onehot_ref, logits_ref):
    p1 = jnp.dot(feat_ref[...], fmu_ref[...], preferred_element_type=jnp.float32)
    p2 = jnp.dot(text_ref[...], tmu_ref[...], preferred_element_type=jnp.float32)
    combined = jnp.concatenate([p1, p2], axis=1)
    logits = jnp.dot(combined, cmu_ref[...], preferred_element_type=jnp.float32)

    eff_temp = jnp.clip(temp_ref[0], 0.5, 5.0)
    logits = jnp.clip(logits / eff_temp, -20.0, 20.0)
    logits_ref[...] = logits

    n = logits.shape[1]
    col = jax.lax.broadcasted_iota(jnp.int32, logits.shape, 1)
    row_max = jnp.max(logits, axis=1, keepdims=True)
    first_arg = jnp.min(jnp.where(logits == row_max, col, n), axis=1,
                        keepdims=True)
    onehot_ref[...] = (col == first_arg).astype(jnp.float32)


def kernel(feature, text_embedding, feature_mu, text_mu, combined_mu,
           temperature):
    tokens, dmodel = feature.shape
    nproj = feature_mu.shape[1]
    nexp = combined_mu.shape[1]
    grid = (tokens // _TILE,)

    onehot, logits = pl.pallas_call(
        _router_body,
        grid=grid,
        in_specs=[
            pl.BlockSpec(memory_space=pltpu.SMEM),
            pl.BlockSpec((_TILE, dmodel), lambda i: (i, 0)),
            pl.BlockSpec((_TILE, dmodel), lambda i: (i, 0)),
            pl.BlockSpec((dmodel, nproj), lambda i: (0, 0)),
            pl.BlockSpec((dmodel, nproj), lambda i: (0, 0)),
            pl.BlockSpec((2 * nproj, nexp), lambda i: (0, 0)),
        ],
        out_specs=[
            pl.BlockSpec((_TILE, nexp), lambda i: (i, 0)),
            pl.BlockSpec((_TILE, nexp), lambda i: (i, 0)),
        ],
        out_shape=[
            jax.ShapeDtypeStruct((tokens, nexp), jnp.float32),
            jax.ShapeDtypeStruct((tokens, nexp), jnp.float32),
        ],
        compiler_params=pltpu.CompilerParams(
            dimension_semantics=("arbitrary",),
        ),
    )(temperature, feature, text_embedding, feature_mu, text_mu, combined_mu)
    return (onehot, logits)
